# baseline (speedup 1.0000x reference)
import jax
import jax.numpy as jnp
from jax import lax
from jax.experimental import pallas as pl
from jax.experimental.pallas import tpu as pltpu

K = 40
ROWS_A = 128
ROWS_B = 128
COARSE_ITERS = 11


def _encode_body(x_ref, wenc_ref, benc_ref, bdec_ref, codes_ref):
    xin = x_ref[...] - bdec_ref[...]
    pre = jnp.dot(xin, wenc_ref[...], preferred_element_type=jnp.float32)
    a = jnp.maximum(pre + benc_ref[...], 0.0)
    rows, d_sae = a.shape
    kf = jnp.float32(K)

    row_max = jnp.max(a, axis=1, keepdims=True)
    hi0 = lax.bitcast_convert_type(
        lax.bitcast_convert_type(row_max, jnp.int32) + 1, jnp.float32
    )

    def coarse_it(_, carry):
        lo, hi, cnt_hi = carry
        mid = 0.5 * (lo + hi)
        cnt = jnp.sum((a >= mid).astype(jnp.float32), axis=1, keepdims=True)
        ge = cnt >= kf
        return (jnp.where(ge, mid, lo), jnp.where(ge, hi, mid),
                jnp.where(ge, cnt_hi, cnt))

    t_lo, t_hi, cnt_above = lax.fori_loop(
        0, COARSE_ITERS, coarse_it,
        (jnp.zeros((rows, 1), jnp.float32), hi0,
         jnp.zeros((rows, 1), jnp.float32)),
    )
    r = kf - cnt_above

    def bucket_max(ub):
        return jnp.max(jnp.where(a < ub, a, -1.0), axis=1, keepdims=True)

    def fine_cond(carry):
        _, r = carry
        return jnp.max(r) > 1.5

    def fine_body(carry):
        ub, r = carry
        m = bucket_max(ub)
        rem = r > 1.5
        return jnp.where(rem, m, ub), r - rem.astype(jnp.float32)

    ub, r = lax.while_loop(fine_cond, fine_body, (t_hi, r))
    v40 = bucket_max(ub)
    thr = jnp.where(v40 > -0.5, v40, t_lo)
    codes_ref[...] = jnp.where(a >= thr, a, 0.0)


def _decode_body(codes_ref, wdec_ref, bdec_ref, recon_ref):
    recon_ref[...] = (
        jnp.dot(codes_ref[...], wdec_ref[...], preferred_element_type=jnp.float32)
        + bdec_ref[...]
    )


@jax.jit
def kernel(x, W_enc, b_enc, W_dec, b_dec):
    B, d_in = x.shape
    d_sae = W_enc.shape[1]

    codes = pl.pallas_call(
        _encode_body,
        grid=(B // ROWS_A,),
        in_specs=[
            pl.BlockSpec((ROWS_A, d_in), lambda i: (i, 0)),
            pl.BlockSpec((d_in, d_sae), lambda i: (0, 0)),
            pl.BlockSpec((1, d_sae), lambda i: (0, 0)),
            pl.BlockSpec((1, d_in), lambda i: (0, 0)),
        ],
        out_specs=pl.BlockSpec((ROWS_A, d_sae), lambda i: (i, 0)),
        out_shape=jax.ShapeDtypeStruct((B, d_sae), jnp.float32),
        compiler_params=pltpu.CompilerParams(
            vmem_limit_bytes=64 * 1024 * 1024,
        ),
    )(x, W_enc, b_enc.reshape(1, d_sae), b_dec.reshape(1, d_in))

    recon = pl.pallas_call(
        _decode_body,
        grid=(B // ROWS_B,),
        in_specs=[
            pl.BlockSpec((ROWS_B, d_sae), lambda i: (i, 0)),
            pl.BlockSpec((d_sae, d_in), lambda i: (0, 0)),
            pl.BlockSpec((1, d_in), lambda i: (0, 0)),
        ],
        out_specs=pl.BlockSpec((ROWS_B, d_in), lambda i: (i, 0)),
        out_shape=jax.ShapeDtypeStruct((B, d_in), jnp.float32),
        compiler_params=pltpu.CompilerParams(
            vmem_limit_bytes=64 * 1024 * 1024,
        ),
    )(codes, W_dec, b_dec.reshape(1, d_in))

    return recon, codes


# speedup vs baseline: 1.0270x; 1.0270x over previous
import jax
import jax.numpy as jnp
from jax import lax
from jax.experimental import pallas as pl
from jax.experimental.pallas import tpu as pltpu

K = 40
ROWS_A = 128
ROWS_B = 256
COARSE_ITERS = 10


def _encode_body(x_ref, wenc_ref, benc_ref, bdec_ref, codes_ref):
    xin = x_ref[...] - bdec_ref[...]
    pre = jnp.dot(xin, wenc_ref[...], preferred_element_type=jnp.float32)
    a = jnp.maximum(pre + benc_ref[...], 0.0)
    rows, d_sae = a.shape
    kf = jnp.float32(K)

    row_max = jnp.max(a, axis=1, keepdims=True)
    hi0 = lax.bitcast_convert_type(
        lax.bitcast_convert_type(row_max, jnp.int32) + 1, jnp.float32
    )

    def coarse_it(_, carry):
        lo, hi, cnt_hi = carry
        mid = 0.5 * (lo + hi)
        cnt = jnp.sum((a >= mid).astype(jnp.float32), axis=1, keepdims=True)
        ge = cnt >= kf
        return (jnp.where(ge, mid, lo), jnp.where(ge, hi, mid),
                jnp.where(ge, cnt_hi, cnt))

    t_lo, t_hi, cnt_above = lax.fori_loop(
        0, COARSE_ITERS, coarse_it,
        (jnp.zeros((rows, 1), jnp.float32), hi0,
         jnp.zeros((rows, 1), jnp.float32)),
    )
    r = kf - cnt_above

    def bucket_max(ub):
        return jnp.max(jnp.where(a < ub, a, -1.0), axis=1, keepdims=True)

    def fine_cond(carry):
        _, r = carry
        return jnp.max(r) > 1.5

    def fine_body(carry):
        ub, r = carry
        m = bucket_max(ub)
        rem = r > 1.5
        return jnp.where(rem, m, ub), r - rem.astype(jnp.float32)

    ub, r = lax.while_loop(fine_cond, fine_body, (t_hi, r))
    v40 = bucket_max(ub)
    thr = jnp.where(v40 > -0.5, v40, t_lo)
    codes_ref[...] = jnp.where(a >= thr, a, 0.0)


def _decode_body(codes_ref, wdec_ref, bdec_ref, recon_ref):
    recon_ref[...] = (
        jnp.dot(codes_ref[...], wdec_ref[...], preferred_element_type=jnp.float32)
        + bdec_ref[...]
    )


@jax.jit
def kernel(x, W_enc, b_enc, W_dec, b_dec):
    B, d_in = x.shape
    d_sae = W_enc.shape[1]

    codes = pl.pallas_call(
        _encode_body,
        grid=(B // ROWS_A,),
        in_specs=[
            pl.BlockSpec((ROWS_A, d_in), lambda i: (i, 0)),
            pl.BlockSpec((d_in, d_sae), lambda i: (0, 0)),
            pl.BlockSpec((1, d_sae), lambda i: (0, 0)),
            pl.BlockSpec((1, d_in), lambda i: (0, 0)),
        ],
        out_specs=pl.BlockSpec((ROWS_A, d_sae), lambda i: (i, 0)),
        out_shape=jax.ShapeDtypeStruct((B, d_sae), jnp.float32),
        compiler_params=pltpu.CompilerParams(
            vmem_limit_bytes=64 * 1024 * 1024,
        ),
    )(x, W_enc, b_enc.reshape(1, d_sae), b_dec.reshape(1, d_in))

    recon = pl.pallas_call(
        _decode_body,
        grid=(B // ROWS_B,),
        in_specs=[
            pl.BlockSpec((ROWS_B, d_sae), lambda i: (i, 0)),
            pl.BlockSpec((d_sae, d_in), lambda i: (0, 0)),
            pl.BlockSpec((1, d_in), lambda i: (0, 0)),
        ],
        out_specs=pl.BlockSpec((ROWS_B, d_in), lambda i: (i, 0)),
        out_shape=jax.ShapeDtypeStruct((B, d_in), jnp.float32),
        compiler_params=pltpu.CompilerParams(
            vmem_limit_bytes=64 * 1024 * 1024,
        ),
    )(codes, W_dec, b_dec.reshape(1, d_in))

    return recon, codes
